# baseline (speedup 1.0000x reference)
import functools

import jax
import jax.numpy as jnp
from jax import lax
from jax.experimental import pallas as pl
from jax.experimental.pallas import tpu as pltpu
from jax.experimental.pallas import tpu_sc as plsc

N = 10000
NP = 10240
E = 160000
EP = 163840
A = 30000
AP = 32768
B = 64
BN = 1024
EBLK = 256
NW = 32
LCH = 128


def _padr(a, n):
    pad = jnp.zeros((n - a.shape[0],) + a.shape[1:], a.dtype)
    return jnp.concatenate([a, pad], axis=0)


def _padi(v, n, fill):
    pad = jnp.full((n - v.shape[0],), fill, jnp.int32)
    return jnp.concatenate([v.astype(jnp.int32), pad], axis=0)


def _sc_mesh():
    return plsc.VectorSubcoreMesh(core_axis_name="c", subcore_axis_name="s")


def _zero_vmem(rows_v, d):
    zv = jnp.zeros((16,), jnp.float32)

    def zr(r, c):
        for l in range(d // 16):
            rows_v[r, pl.ds(l * 16, 16)] = zv
        return c

    lax.fori_loop(0, LCH, zr, 0)


def _sc_gather(table, idx, nch):
    npn, d = table.shape
    epn = idx.shape[0]

    @functools.partial(
        pl.kernel,
        out_type=jax.ShapeDtypeStruct((epn, d), jnp.float32),
        mesh=_sc_mesh(),
        scratch_types=[
            pltpu.VMEM((LCH,), jnp.int32),
            pltpu.VMEM((LCH, d), jnp.float32),
            pltpu.SemaphoreType.DMA,
        ],
    )
    def k(table_hbm, idx_hbm, out_hbm, idx_v, rows_v, sem):
        wid = lax.axis_index("s") * 2 + lax.axis_index("c")
        base = wid * (nch * LCH)

        def body(j, c):
            off = base + j * LCH
            pltpu.sync_copy(idx_hbm.at[pl.ds(off, LCH)], idx_v)
            pltpu.async_copy(table_hbm.at[idx_v], rows_v, sem).wait()
            pltpu.sync_copy(rows_v, out_hbm.at[pl.ds(off, LCH)])
            return c

        lax.fori_loop(0, nch, body, 0)

    return k(table, idx)


def _sc_segsum(rows_or_table, src_idx, dst_idx, nch, indirect_src):
    d = rows_or_table.shape[1]
    nst = NP // 16 // LCH

    scratch = [
        pltpu.VMEM((LCH,), jnp.int32),
        pltpu.VMEM((LCH, d), jnp.float32),
        pltpu.VMEM_SHARED((NP, d), jnp.float32),
        pltpu.SemaphoreType.DMA,
    ]
    if indirect_src:
        scratch.insert(0, pltpu.VMEM((LCH,), jnp.int32))

    @functools.partial(
        pl.kernel,
        out_type=jax.ShapeDtypeStruct((2, NP, d), jnp.float32),
        mesh=_sc_mesh(),
        scratch_types=scratch,
    )
    def k(rows_hbm, src_hbm, dst_hbm, out_hbm, *refs):
        if indirect_src:
            src_v, dst_v, rows_v, agg_sh, sem = refs
        else:
            dst_v, rows_v, agg_sh, sem = refs
            src_v = None
        cid = lax.axis_index("c")
        sid = lax.axis_index("s")
        wid = sid * 2 + cid
        base = wid * (nch * LCH)
        srow = sid * (NP // 16)

        _zero_vmem(rows_v, d)
        for jj in range(nst):
            pltpu.sync_copy(rows_v, agg_sh.at[pl.ds(srow + jj * LCH, LCH)])
        plsc.subcore_barrier()

        def body(j, c):
            off = base + j * LCH
            if indirect_src:
                pltpu.sync_copy(src_hbm.at[pl.ds(off, LCH)], src_v)
                pltpu.async_copy(rows_hbm.at[src_v], rows_v, sem).wait()
            else:
                pltpu.sync_copy(rows_hbm.at[pl.ds(off, LCH)], rows_v)
            pltpu.sync_copy(dst_hbm.at[pl.ds(off, LCH)], dst_v)
            pltpu.sync_copy(rows_v, agg_sh.at[dst_v], add=True)
            return c

        lax.fori_loop(0, nch, body, 0)
        plsc.subcore_barrier()

        for jj in range(nst):
            r0 = srow + jj * LCH
            pltpu.sync_copy(agg_sh.at[pl.ds(r0, LCH)], rows_v)
            pltpu.sync_copy(rows_v, out_hbm.at[cid, pl.ds(r0, LCH)])

    return k(rows_or_table, src_idx, dst_idx)


def _sc_scatter_add(rows, dst_idx, nch):
    return _sc_segsum(rows, dst_idx, dst_idx, nch, indirect_src=False)


def _sc_gather_scatter(table, src_idx, dst_idx, nch):
    return _sc_segsum(table, src_idx, dst_idx, nch, indirect_src=True)


def _elu(v):
    return jnp.where(v > 0, v, jnp.exp(jnp.minimum(v, 0.0)) - 1.0)


def _nnconv_msgs(ea, xs, w1, b1, w2t, b2m, m_in, m_out):
    def body(ea_ref, xs_ref, w1_ref, b1_ref, w2t_ref, b2m_ref, out_ref):
        h = jnp.maximum(
            jnp.dot(ea_ref[...], w1_ref[...], preferred_element_type=jnp.float32)
            + b1_ref[...], 0.0)
        wet = jnp.dot(h, w2t_ref[...], preferred_element_type=jnp.float32)
        we3 = wet.reshape(EBLK, m_out, m_in)
        xsv = xs_ref[...][:, :m_in]
        msg = jnp.sum(we3 * xsv[:, None, :], axis=-1)
        msg = msg + jnp.dot(xsv, b2m_ref[...], preferred_element_type=jnp.float32)
        if m_out < 128:
            msg = jnp.concatenate(
                [msg, jnp.zeros((EBLK, 128 - m_out), jnp.float32)], axis=1)
        out_ref[...] = msg

    return pl.pallas_call(
        body,
        grid=(EP // EBLK,),
        in_specs=[
            pl.BlockSpec((EBLK, 7), lambda i: (i, 0)),
            pl.BlockSpec((EBLK, 128), lambda i: (i, 0)),
            pl.BlockSpec((7, 128), lambda i: (0, 0)),
            pl.BlockSpec((1, 128), lambda i: (0, 0)),
            pl.BlockSpec((128, m_out * m_in), lambda i: (0, 0)),
            pl.BlockSpec((m_in, m_out), lambda i: (0, 0)),
        ],
        out_specs=pl.BlockSpec((EBLK, 128), lambda i: (i, 0)),
        out_shape=jax.ShapeDtypeStruct((EP, 128), jnp.float32),
    )(ea, xs, w1, b1, w2t, b2m)


def _node_affine(x, parts, wr, wn, b, batch3, emit_cat, parts_b=None, wn_b=None):
    di = x.shape[1]
    do = wr.shape[1]
    two_parts = parts_b is not None

    def body(*refs):
        if two_parts:
            (x_ref, p_ref, pb_ref, wr_ref, wn_ref, wnb_ref, b_ref, bt_ref,
             h_ref, pool_ref) = refs
        else:
            x_ref, p_ref, wr_ref, wn_ref, b_ref, bt_ref, h_ref, pool_ref = refs
        agg = p_ref[0] + p_ref[1]
        h = (jnp.dot(x_ref[...], wr_ref[...], preferred_element_type=jnp.float32)
             + jnp.dot(agg, wn_ref[...], preferred_element_type=jnp.float32)
             + b_ref[...])
        if two_parts:
            aggb = pb_ref[0] + pb_ref[1]
            h = h + jnp.dot(aggb, wnb_ref[...], preferred_element_type=jnp.float32)
        h = _elu(h)
        if emit_cat:
            pad_col = (lax.broadcasted_iota(jnp.int32, (BN, 128 - do), 1) == 0
                       ).astype(jnp.float32)
        else:
            pad_col = jnp.zeros((BN, 128 - do), jnp.float32)
        h_ref[...] = jnp.concatenate([h, pad_col], axis=1)
        bt = bt_ref[0]
        oh = (lax.broadcasted_iota(jnp.int32, (B, BN), 0) == bt).astype(jnp.float32)
        pool_blk = jnp.dot(oh, h, preferred_element_type=jnp.float32)

        @pl.when(pl.program_id(0) == 0)
        def _():
            pool_ref[...] = jnp.zeros_like(pool_ref)

        pool_ref[...] += pool_blk

    in_specs = [
        pl.BlockSpec((BN, di), lambda i: (i, 0)),
        pl.BlockSpec((2, BN, 128), lambda i: (0, i, 0)),
    ]
    args = [x, parts]
    if two_parts:
        in_specs.append(pl.BlockSpec((2, BN, 128), lambda i: (0, i, 0)))
        args.append(parts_b)
    in_specs.append(pl.BlockSpec((di, do), lambda i: (0, 0)))
    args.append(wr)
    in_specs.append(pl.BlockSpec((128, do), lambda i: (0, 0)))
    args.append(wn)
    if two_parts:
        in_specs.append(pl.BlockSpec((128, do), lambda i: (0, 0)))
        args.append(wn_b)
    in_specs.append(pl.BlockSpec((1, do), lambda i: (0, 0)))
    args.append(b)
    in_specs.append(pl.BlockSpec((1, 1, BN), lambda i: (i, 0, 0)))
    args.append(batch3)

    return pl.pallas_call(
        body,
        grid=(NP // BN,),
        in_specs=in_specs,
        out_specs=[
            pl.BlockSpec((BN, 128), lambda i: (i, 0)),
            pl.BlockSpec((B, do), lambda i: (0, 0)),
        ],
        out_shape=[
            jax.ShapeDtypeStruct((NP, 128), jnp.float32),
            jax.ShapeDtypeStruct((B, do), jnp.float32),
        ],
    )(*args)


def _pool_concat(parts, iso, outw):
    ni = iso.shape[1]

    def body(p_ref, iso_ref, o_ref):
        s = p_ref[0] + p_ref[1]
        cnt = s[:, 64:65]
        pool = s[:, :64] / jnp.maximum(cnt, 1.0)
        pad = jnp.zeros((BN, outw - 64 - ni), jnp.float32)
        o_ref[...] = jnp.concatenate([pool, iso_ref[...], pad], axis=1)

    return pl.pallas_call(
        body,
        grid=(NP // BN,),
        in_specs=[
            pl.BlockSpec((2, BN, 128), lambda i: (0, i, 0)),
            pl.BlockSpec((BN, ni), lambda i: (i, 0)),
        ],
        out_specs=pl.BlockSpec((BN, outw), lambda i: (i, 0)),
        out_shape=jax.ShapeDtypeStruct((NP, outw), jnp.float32),
    )(parts, iso)


def _mlp(x1, x2, x3, wa, wb, wc, b1, w2, b2, w3, b3):
    def body(x1r, x2r, x3r, war, wbr, wcr, b1r, w2r, b2r, w3r, b3r, o_ref):
        t = _elu(jnp.dot(x1r[...], war[...], preferred_element_type=jnp.float32)
                 + jnp.dot(x2r[...], wbr[...], preferred_element_type=jnp.float32)
                 + jnp.dot(x3r[...], wcr[...], preferred_element_type=jnp.float32)
                 + b1r[...])
        u = _elu(jnp.dot(t, w2r[...], preferred_element_type=jnp.float32) + b2r[...])
        o_ref[...] = jnp.dot(u, w3r[...], preferred_element_type=jnp.float32) + b3r[...]

    return pl.pallas_call(
        body,
        out_shape=jax.ShapeDtypeStruct((B, 1), jnp.float32),
    )(x1, x2, x3, wa, wb, wc, b1, w2, b2, w3, b3)


def _w2perm(w2, m_in, m_out):
    return w2.reshape(128, m_in, m_out).transpose(0, 2, 1).reshape(128, m_out * m_in)


def kernel(x, edge_attr, iso_type_2, iso_type_3, params, edge_index, batch,
           assignment_index_2, edge_index_2, batch_2,
           assignment_index_3, edge_index_3, batch_3):
    p = params
    f32 = jnp.float32

    x_p = _padr(x.astype(f32), NP)
    ea_p = _padr(edge_attr.astype(f32), EP)
    iso2_p = _padr(iso_type_2.astype(f32), NP)
    iso3_p = _padr(iso_type_3.astype(f32), NP)

    src1 = _padi(edge_index[0], EP, 0)
    dst1 = _padi(edge_index[1], EP, N)
    src2 = _padi(edge_index_2[0], EP, 0)
    dst2 = _padi(edge_index_2[1], EP, N)
    src3 = _padi(edge_index_3[0], EP, 0)
    dst3 = _padi(edge_index_3[1], EP, N)
    a2s = _padi(assignment_index_2[0], AP, 0)
    a2d = _padi(assignment_index_2[1], AP, N)
    a3s = _padi(assignment_index_3[0], AP, 0)
    a3d = _padi(assignment_index_3[1], AP, N)
    bt1 = _padi(batch, NP, B).reshape(NP // BN, 1, BN)
    bt2 = _padi(batch_2, NP, B).reshape(NP // BN, 1, BN)
    bt3 = _padi(batch_3, NP, B).reshape(NP // BN, 1, BN)

    def eyep(m):
        return jnp.concatenate([jnp.eye(m, dtype=f32),
                                jnp.zeros((128 - m, m), f32)], axis=0)

    row128 = lambda v: v.reshape(1, -1).astype(f32)

    w2t1 = _w2perm(p['nn1_W2'], 128, 32)
    w2t2 = _w2perm(p['nn2_W2'], 32, 64)
    w2t3 = _w2perm(p['nn3_W2'], 64, 64)
    b2m1 = p['nn1_b2'].reshape(128, 32)
    b2m2 = p['nn2_b2'].reshape(32, 64)
    b2m3 = p['nn3_b2'].reshape(64, 64)

    padw = lambda w, r: jnp.concatenate(
        [w.astype(f32), jnp.zeros((r - w.shape[0], w.shape[1]), f32)], axis=0)
    w4r, w4n = padw(p['W4r'], 128), padw(p['W4n'], 128)
    w6r, w6n = padw(p['W6r'], 256), padw(p['W6n'], 256)
    w5r, w5n = padw(p['W5r'], 128), padw(p['W5n'], 128)
    w7r, w7n = padw(p['W7r'], 128), padw(p['W7n'], 128)
    root2p, root3p = padw(p['root2'], 128), padw(p['root3'], 128)

    fc1s = p['fc1W'][:192] + p['fc1W'][192:]
    fca, fcb, fcc = fc1s[0:64], fc1s[64:128], fc1s[128:192]

    ech = EP // NW // LCH
    ach = AP // NW // LCH
    xs1 = _sc_gather(x_p, src1, ech)
    msg1 = _nnconv_msgs(ea_p, xs1, p['nn1_W1'], row128(p['nn1_b1']), w2t1, b2m1, 128, 32)
    parts1 = _sc_scatter_add(msg1, dst1, ech)
    h1, _ = _node_affine(x_p, parts1, p['root1'], eyep(32), row128(p['bias1']), bt1, False)

    xs2 = _sc_gather(h1, src1, ech)
    msg2 = _nnconv_msgs(ea_p, xs2, p['nn2_W1'], row128(p['nn2_b1']), w2t2, b2m2, 32, 64)
    parts2 = _sc_scatter_add(msg2, dst1, ech)
    h2, _ = _node_affine(h1, parts2, root2p, eyep(64), row128(p['bias2']), bt1, False)

    xs3 = _sc_gather(h2, src1, ech)
    msg3 = _nnconv_msgs(ea_p, xs3, p['nn3_W1'], row128(p['nn3_b1']), w2t3, b2m3, 64, 64)
    parts3 = _sc_scatter_add(msg3, dst1, ech)
    h3cat, x1 = _node_affine(h2, parts3, root3p, eyep(64), row128(p['bias3']), bt1, True)

    pp2 = _sc_gather_scatter(h3cat, a2s, a2d, ach)
    h2cat = _pool_concat(pp2, iso2_p, 128)
    g4 = _sc_gather_scatter(h2cat, src2, dst2, ech)
    h4, _ = _node_affine(h2cat, g4, w4r, w4n, row128(p['b4']), bt2, False)
    g5 = _sc_gather_scatter(h4, src2, dst2, ech)
    h5, x2 = _node_affine(h4, g5, w5r, w5n, row128(p['b5']), bt2, False)

    pp3 = _sc_gather_scatter(h3cat, a3s, a3d, ach)
    h3cat2 = _pool_concat(pp3, iso3_p, 256)
    h3a, h3b = h3cat2[:, :128], h3cat2[:, 128:]
    g6a = _sc_gather_scatter(h3a, src3, dst3, ech)
    g6b = _sc_gather_scatter(h3b, src3, dst3, ech)
    h6, _ = _node_affine(h3cat2, g6a, w6r, w6n[:128], row128(p['b6']), bt3,
                         False, parts_b=g6b, wn_b=w6n[128:])
    g7 = _sc_gather_scatter(h6, src3, dst3, ech)
    h7, x3 = _node_affine(h6, g7, w7r, w7n, row128(p['b7']), bt3, False)

    o = _mlp(x1, x2, x3, fca, fcb, fcc, row128(p['fc1b']),
             p['fc2W'], row128(p['fc2b']), p['fc3W'], p['fc3b'].reshape(1, 1))
    return o.reshape(-1)


# speedup vs baseline: 2.2540x; 2.2540x over previous
import functools

import jax
import jax.numpy as jnp
from jax import lax
from jax.experimental import pallas as pl
from jax.experimental.pallas import tpu as pltpu
from jax.experimental.pallas import tpu_sc as plsc

N = 10000
NP = 10240
E = 160000
EP = 163840
A = 30000
AP = 32768
B = 64
BN = 1024
EBLK = 256
NW = 32
LCH = 128


def _padr(a, n):
    pad = jnp.zeros((n - a.shape[0],) + a.shape[1:], a.dtype)
    return jnp.concatenate([a, pad], axis=0)


def _padi(v, n, fill):
    pad = jnp.full((n - v.shape[0],), fill, jnp.int32)
    return jnp.concatenate([v.astype(jnp.int32), pad], axis=0)


def _sc_mesh():
    return plsc.VectorSubcoreMesh(core_axis_name="c", subcore_axis_name="s")


def _zero_vmem(rows_v, d):
    zv = jnp.zeros((16,), jnp.float32)

    def zr(r, c):
        for l in range(d // 16):
            rows_v[r, pl.ds(l * 16, 16)] = zv
        return c

    lax.fori_loop(0, LCH, zr, 0)


def _sc_gather(table, idx, nch):
    npn, d = table.shape
    epn = idx.shape[0]

    @functools.partial(
        pl.kernel,
        out_type=jax.ShapeDtypeStruct((epn, d), jnp.float32),
        mesh=_sc_mesh(),
        scratch_types=[
            pltpu.VMEM((LCH,), jnp.int32),
            pltpu.VMEM((LCH, d), jnp.float32),
            pltpu.SemaphoreType.DMA,
        ],
    )
    def k(table_hbm, idx_hbm, out_hbm, idx_v, rows_v, sem):
        wid = lax.axis_index("s") * 2 + lax.axis_index("c")
        base = wid * (nch * LCH)

        def body(j, c):
            off = base + j * LCH
            pltpu.sync_copy(idx_hbm.at[pl.ds(off, LCH)], idx_v)
            pltpu.async_copy(table_hbm.at[idx_v], rows_v, sem).wait()
            pltpu.sync_copy(rows_v, out_hbm.at[pl.ds(off, LCH)])
            return c

        lax.fori_loop(0, nch, body, 0)

    return k(table, idx)


def _sc_segsum(rows_or_table, src_idx, dst_idx, nch, indirect_src):
    d = rows_or_table.shape[1]
    nst = NP // 16 // LCH

    scratch = [
        pltpu.VMEM((LCH,), jnp.int32),
        pltpu.VMEM((LCH, d), jnp.float32),
        pltpu.VMEM_SHARED((NP, d), jnp.float32),
        pltpu.SemaphoreType.DMA,
    ]
    if indirect_src:
        scratch.insert(0, pltpu.VMEM((LCH,), jnp.int32))

    @functools.partial(
        pl.kernel,
        out_type=jax.ShapeDtypeStruct((2, NP, d), jnp.float32),
        mesh=_sc_mesh(),
        scratch_types=scratch,
    )
    def k(rows_hbm, src_hbm, dst_hbm, out_hbm, *refs):
        if indirect_src:
            src_v, dst_v, rows_v, agg_sh, sem = refs
        else:
            dst_v, rows_v, agg_sh, sem = refs
            src_v = None
        cid = lax.axis_index("c")
        sid = lax.axis_index("s")
        wid = sid * 2 + cid
        base = wid * (nch * LCH)
        srow = sid * (NP // 16)

        _zero_vmem(rows_v, d)
        for jj in range(nst):
            pltpu.sync_copy(rows_v, agg_sh.at[pl.ds(srow + jj * LCH, LCH)])
        plsc.subcore_barrier()

        def body(j, c):
            off = base + j * LCH
            if indirect_src:
                pltpu.sync_copy(src_hbm.at[pl.ds(off, LCH)], src_v)
                pltpu.async_copy(rows_hbm.at[src_v], rows_v, sem).wait()
            else:
                pltpu.sync_copy(rows_hbm.at[pl.ds(off, LCH)], rows_v)
            pltpu.sync_copy(dst_hbm.at[pl.ds(off, LCH)], dst_v)
            pltpu.sync_copy(rows_v, agg_sh.at[dst_v], add=True)
            return c

        lax.fori_loop(0, nch, body, 0)
        plsc.subcore_barrier()

        for jj in range(nst):
            r0 = srow + jj * LCH
            pltpu.sync_copy(agg_sh.at[pl.ds(r0, LCH)], rows_v)
            pltpu.sync_copy(rows_v, out_hbm.at[cid, pl.ds(r0, LCH)])

    return k(rows_or_table, src_idx, dst_idx)


def _sc_scatter_add(rows, dst_idx, nch):
    return _sc_segsum(rows, dst_idx, dst_idx, nch, indirect_src=False)


def _sc_gather_scatter(table, src_idx, dst_idx, nch):
    return _sc_segsum(table, src_idx, dst_idx, nch, indirect_src=True)


def _elu(v):
    return jnp.where(v > 0, v, jnp.exp(jnp.minimum(v, 0.0)) - 1.0)


def _nnconv_msgs(ea, xs, w1, b1, w2p, b2m, sel, m_in, m_out):
    def body(ea_ref, xs_ref, w1_ref, b1_ref, w2p_ref, b2m_ref, sel_ref, out_ref):
        h = jnp.maximum(
            jnp.dot(ea_ref[...], w1_ref[...], preferred_element_type=jnp.float32)
            + b1_ref[...], 0.0)
        xs_m = xs_ref[...] if m_in == 128 else xs_ref[...][:, :m_in]
        qt = jnp.dot(xs_m, w2p_ref[...], preferred_element_type=jnp.float32)
        r = qt * jnp.tile(h, (1, m_out))
        msg = jnp.dot(r, sel_ref[...], preferred_element_type=jnp.float32)
        msg = msg + jnp.dot(xs_m, b2m_ref[...], preferred_element_type=jnp.float32)
        out_ref[...] = msg

    return pl.pallas_call(
        body,
        grid=(EP // EBLK,),
        in_specs=[
            pl.BlockSpec((EBLK, 7), lambda i: (i, 0)),
            pl.BlockSpec((EBLK, 128), lambda i: (i, 0)),
            pl.BlockSpec((7, 128), lambda i: (0, 0)),
            pl.BlockSpec((1, 128), lambda i: (0, 0)),
            pl.BlockSpec((m_in, m_out * 128), lambda i: (0, 0)),
            pl.BlockSpec((m_in, m_out), lambda i: (0, 0)),
            pl.BlockSpec((m_out * 128, m_out), lambda i: (0, 0)),
        ],
        out_specs=pl.BlockSpec((EBLK, m_out), lambda i: (i, 0)),
        out_shape=jax.ShapeDtypeStruct((EP, m_out), jnp.float32),
    )(ea, xs, w1, b1, w2p, b2m, sel)


def _node_affine(x, parts, wr, wn, b, batch3, emit_cat, parts_b=None, wn_b=None):
    di = x.shape[1]
    dm = parts.shape[2]
    do = wr.shape[1]
    two_parts = parts_b is not None

    def body(*refs):
        if two_parts:
            (x_ref, p_ref, pb_ref, wr_ref, wn_ref, wnb_ref, b_ref, bt_ref,
             h_ref, pool_ref) = refs
        else:
            x_ref, p_ref, wr_ref, wn_ref, b_ref, bt_ref, h_ref, pool_ref = refs
        agg = p_ref[0] + p_ref[1]
        h = (jnp.dot(x_ref[...], wr_ref[...], preferred_element_type=jnp.float32)
             + jnp.dot(agg, wn_ref[...], preferred_element_type=jnp.float32)
             + b_ref[...])
        if two_parts:
            aggb = pb_ref[0] + pb_ref[1]
            h = h + jnp.dot(aggb, wnb_ref[...], preferred_element_type=jnp.float32)
        h = _elu(h)
        if emit_cat:
            pad_col = (lax.broadcasted_iota(jnp.int32, (BN, 128 - do), 1) == 0
                       ).astype(jnp.float32)
        else:
            pad_col = jnp.zeros((BN, 128 - do), jnp.float32)
        h_ref[...] = jnp.concatenate([h, pad_col], axis=1)
        bt = bt_ref[0]
        oh = (lax.broadcasted_iota(jnp.int32, (B, BN), 0) == bt).astype(jnp.float32)
        pool_blk = jnp.dot(oh, h, preferred_element_type=jnp.float32)

        @pl.when(pl.program_id(0) == 0)
        def _():
            pool_ref[...] = jnp.zeros_like(pool_ref)

        pool_ref[...] += pool_blk

    in_specs = [
        pl.BlockSpec((BN, di), lambda i: (i, 0)),
        pl.BlockSpec((2, BN, dm), lambda i: (0, i, 0)),
    ]
    args = [x, parts]
    if two_parts:
        in_specs.append(pl.BlockSpec((2, BN, 128), lambda i: (0, i, 0)))
        args.append(parts_b)
    in_specs.append(pl.BlockSpec((di, do), lambda i: (0, 0)))
    args.append(wr)
    in_specs.append(pl.BlockSpec((dm, do), lambda i: (0, 0)))
    args.append(wn)
    if two_parts:
        in_specs.append(pl.BlockSpec((128, do), lambda i: (0, 0)))
        args.append(wn_b)
    in_specs.append(pl.BlockSpec((1, do), lambda i: (0, 0)))
    args.append(b)
    in_specs.append(pl.BlockSpec((1, 1, BN), lambda i: (i, 0, 0)))
    args.append(batch3)

    return pl.pallas_call(
        body,
        grid=(NP // BN,),
        in_specs=in_specs,
        out_specs=[
            pl.BlockSpec((BN, 128), lambda i: (i, 0)),
            pl.BlockSpec((B, do), lambda i: (0, 0)),
        ],
        out_shape=[
            jax.ShapeDtypeStruct((NP, 128), jnp.float32),
            jax.ShapeDtypeStruct((B, do), jnp.float32),
        ],
    )(*args)


def _pool_concat(parts, iso, outw):
    ni = iso.shape[1]

    def body(p_ref, iso_ref, o_ref):
        s = p_ref[0] + p_ref[1]
        cnt = s[:, 64:65]
        pool = s[:, :64] / jnp.maximum(cnt, 1.0)
        pad = jnp.zeros((BN, outw - 64 - ni), jnp.float32)
        o_ref[...] = jnp.concatenate([pool, iso_ref[...], pad], axis=1)

    return pl.pallas_call(
        body,
        grid=(NP // BN,),
        in_specs=[
            pl.BlockSpec((2, BN, 128), lambda i: (0, i, 0)),
            pl.BlockSpec((BN, ni), lambda i: (i, 0)),
        ],
        out_specs=pl.BlockSpec((BN, outw), lambda i: (i, 0)),
        out_shape=jax.ShapeDtypeStruct((NP, outw), jnp.float32),
    )(parts, iso)


def _mlp(x1, x2, x3, wa, wb, wc, b1, w2, b2, w3, b3):
    def body(x1r, x2r, x3r, war, wbr, wcr, b1r, w2r, b2r, w3r, b3r, o_ref):
        t = _elu(jnp.dot(x1r[...], war[...], preferred_element_type=jnp.float32)
                 + jnp.dot(x2r[...], wbr[...], preferred_element_type=jnp.float32)
                 + jnp.dot(x3r[...], wcr[...], preferred_element_type=jnp.float32)
                 + b1r[...])
        u = _elu(jnp.dot(t, w2r[...], preferred_element_type=jnp.float32) + b2r[...])
        o_ref[...] = jnp.dot(u, w3r[...], preferred_element_type=jnp.float32) + b3r[...]

    return pl.pallas_call(
        body,
        out_shape=jax.ShapeDtypeStruct((B, 1), jnp.float32),
    )(x1, x2, x3, wa, wb, wc, b1, w2, b2, w3, b3)


def _w2perm(w2, m_in, m_out):
    return w2.reshape(128, m_in, m_out).transpose(1, 2, 0).reshape(m_in, m_out * 128)


def kernel(x, edge_attr, iso_type_2, iso_type_3, params, edge_index, batch,
           assignment_index_2, edge_index_2, batch_2,
           assignment_index_3, edge_index_3, batch_3):
    p = params
    f32 = jnp.float32

    x_p = _padr(x.astype(f32), NP)
    ea_p = _padr(edge_attr.astype(f32), EP)
    iso2_p = _padr(iso_type_2.astype(f32), NP)
    iso3_p = _padr(iso_type_3.astype(f32), NP)

    src1 = _padi(edge_index[0], EP, 0)
    dst1 = _padi(edge_index[1], EP, N)
    src2 = _padi(edge_index_2[0], EP, 0)
    dst2 = _padi(edge_index_2[1], EP, N)
    src3 = _padi(edge_index_3[0], EP, 0)
    dst3 = _padi(edge_index_3[1], EP, N)
    a2s = _padi(assignment_index_2[0], AP, 0)
    a2d = _padi(assignment_index_2[1], AP, N)
    a3s = _padi(assignment_index_3[0], AP, 0)
    a3d = _padi(assignment_index_3[1], AP, N)
    bt1 = _padi(batch, NP, B).reshape(NP // BN, 1, BN)
    bt2 = _padi(batch_2, NP, B).reshape(NP // BN, 1, BN)
    bt3 = _padi(batch_3, NP, B).reshape(NP // BN, 1, BN)

    def eyep(m):
        return jnp.concatenate([jnp.eye(m, dtype=f32),
                                jnp.zeros((128 - m, m), f32)], axis=0)

    row128 = lambda v: v.reshape(1, -1).astype(f32)

    w2p1 = _w2perm(p['nn1_W2'], 128, 32)
    w2p2 = _w2perm(p['nn2_W2'], 32, 64)
    w2p3 = _w2perm(p['nn3_W2'], 64, 64)
    b2m1 = p['nn1_b2'].reshape(128, 32)
    b2m2 = p['nn2_b2'].reshape(32, 64)
    b2m3 = p['nn3_b2'].reshape(64, 64)
    sel32 = jnp.kron(jnp.eye(32, dtype=f32), jnp.ones((128, 1), f32))
    sel64 = jnp.kron(jnp.eye(64, dtype=f32), jnp.ones((128, 1), f32))

    padw = lambda w, r: jnp.concatenate(
        [w.astype(f32), jnp.zeros((r - w.shape[0], w.shape[1]), f32)], axis=0)
    w4r, w4n = padw(p['W4r'], 128), padw(p['W4n'], 128)
    w6r, w6n = padw(p['W6r'], 256), padw(p['W6n'], 256)
    w5r, w5n = padw(p['W5r'], 128), padw(p['W5n'], 128)
    w7r, w7n = padw(p['W7r'], 128), padw(p['W7n'], 128)
    root2p, root3p = padw(p['root2'], 128), padw(p['root3'], 128)

    fc1s = p['fc1W'][:192] + p['fc1W'][192:]
    fca, fcb, fcc = fc1s[0:64], fc1s[64:128], fc1s[128:192]

    ech = EP // NW // LCH
    ach = AP // NW // LCH
    xs1 = _sc_gather(x_p, src1, ech)
    msg1 = _nnconv_msgs(ea_p, xs1, p['nn1_W1'], row128(p['nn1_b1']), w2p1, b2m1, sel32, 128, 32)
    parts1 = _sc_scatter_add(msg1, dst1, ech)
    h1, _ = _node_affine(x_p, parts1, p['root1'], jnp.eye(32, dtype=f32),
                         row128(p['bias1']), bt1, False)

    xs2 = _sc_gather(h1, src1, ech)
    msg2 = _nnconv_msgs(ea_p, xs2, p['nn2_W1'], row128(p['nn2_b1']), w2p2, b2m2, sel64, 32, 64)
    parts2 = _sc_scatter_add(msg2, dst1, ech)
    h2, _ = _node_affine(h1, parts2, root2p, jnp.eye(64, dtype=f32),
                         row128(p['bias2']), bt1, False)

    xs3 = _sc_gather(h2, src1, ech)
    msg3 = _nnconv_msgs(ea_p, xs3, p['nn3_W1'], row128(p['nn3_b1']), w2p3, b2m3, sel64, 64, 64)
    parts3 = _sc_scatter_add(msg3, dst1, ech)
    h3cat, x1 = _node_affine(h2, parts3, root3p, jnp.eye(64, dtype=f32),
                             row128(p['bias3']), bt1, True)

    pp2 = _sc_gather_scatter(h3cat, a2s, a2d, ach)
    h2cat = _pool_concat(pp2, iso2_p, 128)
    g4 = _sc_gather_scatter(h2cat, src2, dst2, ech)
    h4, _ = _node_affine(h2cat, g4, w4r, w4n, row128(p['b4']), bt2, False)
    g5 = _sc_gather_scatter(h4, src2, dst2, ech)
    h5, x2 = _node_affine(h4, g5, w5r, w5n, row128(p['b5']), bt2, False)

    pp3 = _sc_gather_scatter(h3cat, a3s, a3d, ach)
    h3cat2 = _pool_concat(pp3, iso3_p, 256)
    h3a, h3b = h3cat2[:, :128], h3cat2[:, 128:]
    g6a = _sc_gather_scatter(h3a, src3, dst3, ech)
    g6b = _sc_gather_scatter(h3b, src3, dst3, ech)
    h6, _ = _node_affine(h3cat2, g6a, w6r, w6n[:128], row128(p['b6']), bt3,
                         False, parts_b=g6b, wn_b=w6n[128:])
    g7 = _sc_gather_scatter(h6, src3, dst3, ech)
    h7, x3 = _node_affine(h6, g7, w7r, w7n, row128(p['b7']), bt3, False)

    o = _mlp(x1, x2, x3, fca, fcb, fcc, row128(p['fc1b']),
             p['fc2W'], row128(p['fc2b']), p['fc3W'], p['fc3b'].reshape(1, 1))
    return o.reshape(-1)


# speedup vs baseline: 2.5006x; 1.1094x over previous
import functools

import jax
import jax.numpy as jnp
from jax import lax
from jax.experimental import pallas as pl
from jax.experimental.pallas import tpu as pltpu
from jax.experimental.pallas import tpu_sc as plsc

N = 10000
NP = 10240
E = 160000
EP = 163840
A = 30000
AP = 32768
B = 64
BN = 1024
EBLK = 256
NW = 32
LCH = 128


def _padr(a, n):
    pad = jnp.zeros((n - a.shape[0],) + a.shape[1:], a.dtype)
    return jnp.concatenate([a, pad], axis=0)


def _padi(v, n, fill):
    pad = jnp.full((n - v.shape[0],), fill, jnp.int32)
    return jnp.concatenate([v.astype(jnp.int32), pad], axis=0)


def _sc_mesh():
    return plsc.VectorSubcoreMesh(core_axis_name="c", subcore_axis_name="s")


def _zero_vmem(rows_v, d):
    zv = jnp.zeros((16,), jnp.float32)

    def zr(r, c):
        for l in range(d // 16):
            rows_v[r, pl.ds(l * 16, 16)] = zv
        return c

    lax.fori_loop(0, LCH, zr, 0)


def _sc_gather(table, idx, nch):
    npn, d = table.shape
    epn = idx.shape[0]

    @functools.partial(
        pl.kernel,
        out_type=jax.ShapeDtypeStruct((epn, d), jnp.float32),
        mesh=_sc_mesh(),
        scratch_types=[
            pltpu.VMEM((LCH,), jnp.int32),
            pltpu.VMEM((LCH, d), jnp.float32),
            pltpu.SemaphoreType.DMA,
        ],
    )
    def k(table_hbm, idx_hbm, out_hbm, idx_v, rows_v, sem):
        wid = lax.axis_index("s") * 2 + lax.axis_index("c")
        base = wid * (nch * LCH)

        def body(j, c):
            off = base + j * LCH
            pltpu.sync_copy(idx_hbm.at[pl.ds(off, LCH)], idx_v)
            pltpu.async_copy(table_hbm.at[idx_v], rows_v, sem).wait()
            pltpu.sync_copy(rows_v, out_hbm.at[pl.ds(off, LCH)])
            return c

        lax.fori_loop(0, nch, body, 0)

    return k(table, idx)


def _sc_segsum(rows_or_table, src_idx, dst_idx, nch, indirect_src):
    d = rows_or_table.shape[1]
    nst = NP // 16 // LCH

    scratch = [
        pltpu.VMEM((LCH,), jnp.int32),
        pltpu.VMEM((LCH, d), jnp.float32),
        pltpu.VMEM_SHARED((NP, d), jnp.float32),
        pltpu.SemaphoreType.DMA,
    ]
    if indirect_src:
        scratch.insert(0, pltpu.VMEM((LCH,), jnp.int32))

    @functools.partial(
        pl.kernel,
        out_type=jax.ShapeDtypeStruct((2, NP, d), jnp.float32),
        mesh=_sc_mesh(),
        scratch_types=scratch,
    )
    def k(rows_hbm, src_hbm, dst_hbm, out_hbm, *refs):
        if indirect_src:
            src_v, dst_v, rows_v, agg_sh, sem = refs
        else:
            dst_v, rows_v, agg_sh, sem = refs
            src_v = None
        cid = lax.axis_index("c")
        sid = lax.axis_index("s")
        wid = sid * 2 + cid
        base = wid * (nch * LCH)
        srow = sid * (NP // 16)

        _zero_vmem(rows_v, d)
        for jj in range(nst):
            pltpu.sync_copy(rows_v, agg_sh.at[pl.ds(srow + jj * LCH, LCH)])
        plsc.subcore_barrier()

        def body(j, c):
            off = base + j * LCH
            if indirect_src:
                pltpu.sync_copy(src_hbm.at[pl.ds(off, LCH)], src_v)
                pltpu.async_copy(rows_hbm.at[src_v], rows_v, sem).wait()
            else:
                pltpu.sync_copy(rows_hbm.at[pl.ds(off, LCH)], rows_v)
            pltpu.sync_copy(dst_hbm.at[pl.ds(off, LCH)], dst_v)
            pltpu.sync_copy(rows_v, agg_sh.at[dst_v], add=True)
            return c

        lax.fori_loop(0, nch, body, 0)
        plsc.subcore_barrier()

        for jj in range(nst):
            r0 = srow + jj * LCH
            pltpu.sync_copy(agg_sh.at[pl.ds(r0, LCH)], rows_v)
            pltpu.sync_copy(rows_v, out_hbm.at[cid, pl.ds(r0, LCH)])

    return k(rows_or_table, src_idx, dst_idx)


def _sc_scatter_add(rows, dst_idx, nch):
    return _sc_segsum(rows, dst_idx, dst_idx, nch, indirect_src=False)


def _sc_gather_scatter(table, src_idx, dst_idx, nch):
    return _sc_segsum(table, src_idx, dst_idx, nch, indirect_src=True)


def _elu(v):
    return jnp.where(v > 0, v, jnp.exp(jnp.minimum(v, 0.0)) - 1.0)


def _nnconv_msgs(ea, xs, w1, b1, w2t, b2m, sel, tilemat, m_in, m_out):
    bf = jnp.bfloat16
    f32 = jnp.float32

    def body(ea_ref, xs_ref, w1_ref, b1_ref, w2t_ref, b2p_ref, sel_ref,
             *rest):
        tm_ref, out_ref = (rest if m_in < 128 else (None, rest[0]))
        h = jnp.maximum(
            jnp.dot(ea_ref[...].astype(bf), w1_ref[...].astype(bf),
                    preferred_element_type=f32)
            + b1_ref[...], 0.0)
        xs_m = xs_ref[...] if m_in == 128 else xs_ref[...][:, :m_in]
        wet = jnp.dot(h.astype(bf), w2t_ref[...].astype(bf),
                      preferred_element_type=f32) + b2p_ref[...]
        wet = wet.astype(bf).astype(f32)
        xs_b = xs_m.astype(bf).astype(f32)
        if m_in == 128:
            xst = jnp.tile(xs_b, (1, m_out))
        else:
            xst = jnp.dot(xs_b, tm_ref[...], preferred_element_type=f32)
        r = wet * xst
        msg = jnp.dot(r, sel_ref[...], preferred_element_type=f32)
        if m_out < 128:
            msg = jnp.concatenate(
                [msg, jnp.zeros((EBLK, 128 - m_out), f32)], axis=1)
        out_ref[...] = msg

    in_specs = [
        pl.BlockSpec((EBLK, 7), lambda i: (i, 0)),
        pl.BlockSpec((EBLK, 128), lambda i: (i, 0)),
        pl.BlockSpec((7, 128), lambda i: (0, 0)),
        pl.BlockSpec((1, 128), lambda i: (0, 0)),
        pl.BlockSpec((128, m_out * m_in), lambda i: (0, 0)),
        pl.BlockSpec((1, m_out * m_in), lambda i: (0, 0)),
        pl.BlockSpec((m_out * m_in, m_out), lambda i: (0, 0)),
    ]
    args = [ea, xs, w1, b1, w2t, b2m, sel]
    if m_in < 128:
        in_specs.append(pl.BlockSpec((m_in, m_out * m_in), lambda i: (0, 0)))
        args.append(tilemat)

    return pl.pallas_call(
        body,
        grid=(EP // EBLK,),
        in_specs=in_specs,
        out_specs=pl.BlockSpec((EBLK, 128), lambda i: (i, 0)),
        out_shape=jax.ShapeDtypeStruct((EP, 128), jnp.float32),
    )(*args)


def _node_affine(x, parts, wr, wn, b, batch3, emit_cat, parts_b=None, wn_b=None,
                 cast_wn=False):
    di = x.shape[1]
    dm = parts.shape[2]
    do = wr.shape[1]
    two_parts = parts_b is not None

    def body(*refs):
        if two_parts:
            (x_ref, p_ref, pb_ref, wr_ref, wn_ref, wnb_ref, b_ref, bt_ref,
             h_ref, pool_ref) = refs
        else:
            x_ref, p_ref, wr_ref, wn_ref, b_ref, bt_ref, h_ref, pool_ref = refs
        bf = jnp.bfloat16
        f32 = jnp.float32
        agg = p_ref[0] + p_ref[1]
        if cast_wn:
            aggd = jnp.dot(agg.astype(bf), wn_ref[...].astype(bf),
                           preferred_element_type=f32)
        else:
            aggd = jnp.dot(agg, wn_ref[...], preferred_element_type=f32)
        h = (jnp.dot(x_ref[...].astype(bf), wr_ref[...].astype(bf),
                     preferred_element_type=f32)
             + aggd + b_ref[...])
        if two_parts:
            aggb = pb_ref[0] + pb_ref[1]
            h = h + jnp.dot(aggb.astype(bf), wnb_ref[...].astype(bf),
                            preferred_element_type=f32)
        h = _elu(h)
        if emit_cat:
            pad_col = (lax.broadcasted_iota(jnp.int32, (BN, 128 - do), 1) == 0
                       ).astype(jnp.float32)
        else:
            pad_col = jnp.zeros((BN, 128 - do), jnp.float32)
        h_ref[...] = jnp.concatenate([h, pad_col], axis=1)
        bt = bt_ref[0]
        oh = (lax.broadcasted_iota(jnp.int32, (B, BN), 0) == bt).astype(jnp.float32)
        pool_blk = jnp.dot(oh, h, preferred_element_type=jnp.float32)

        @pl.when(pl.program_id(0) == 0)
        def _():
            pool_ref[...] = jnp.zeros_like(pool_ref)

        pool_ref[...] += pool_blk

    in_specs = [
        pl.BlockSpec((BN, di), lambda i: (i, 0)),
        pl.BlockSpec((2, BN, dm), lambda i: (0, i, 0)),
    ]
    args = [x, parts]
    if two_parts:
        in_specs.append(pl.BlockSpec((2, BN, 128), lambda i: (0, i, 0)))
        args.append(parts_b)
    in_specs.append(pl.BlockSpec((di, do), lambda i: (0, 0)))
    args.append(wr)
    in_specs.append(pl.BlockSpec((dm, do), lambda i: (0, 0)))
    args.append(wn)
    if two_parts:
        in_specs.append(pl.BlockSpec((128, do), lambda i: (0, 0)))
        args.append(wn_b)
    in_specs.append(pl.BlockSpec((1, do), lambda i: (0, 0)))
    args.append(b)
    in_specs.append(pl.BlockSpec((1, 1, BN), lambda i: (i, 0, 0)))
    args.append(batch3)

    return pl.pallas_call(
        body,
        grid=(NP // BN,),
        in_specs=in_specs,
        out_specs=[
            pl.BlockSpec((BN, 128), lambda i: (i, 0)),
            pl.BlockSpec((B, do), lambda i: (0, 0)),
        ],
        out_shape=[
            jax.ShapeDtypeStruct((NP, 128), jnp.float32),
            jax.ShapeDtypeStruct((B, do), jnp.float32),
        ],
    )(*args)


def _pool_concat(parts, iso, outw):
    ni = iso.shape[1]

    def body(p_ref, iso_ref, o_ref):
        s = p_ref[0] + p_ref[1]
        cnt = s[:, 64:65]
        pool = s[:, :64] / jnp.maximum(cnt, 1.0)
        pad = jnp.zeros((BN, outw - 64 - ni), jnp.float32)
        o_ref[...] = jnp.concatenate([pool, iso_ref[...], pad], axis=1)

    return pl.pallas_call(
        body,
        grid=(NP // BN,),
        in_specs=[
            pl.BlockSpec((2, BN, 128), lambda i: (0, i, 0)),
            pl.BlockSpec((BN, ni), lambda i: (i, 0)),
        ],
        out_specs=pl.BlockSpec((BN, outw), lambda i: (i, 0)),
        out_shape=jax.ShapeDtypeStruct((NP, outw), jnp.float32),
    )(parts, iso)


def _mlp(x1, x2, x3, wa, wb, wc, b1, w2, b2, w3, b3):
    def body(x1r, x2r, x3r, war, wbr, wcr, b1r, w2r, b2r, w3r, b3r, o_ref):
        f32 = jnp.float32
        rt = lambda a: a.astype(jnp.bfloat16).astype(f32)
        t = _elu(jnp.dot(rt(x1r[...]), war[...], preferred_element_type=f32)
                 + jnp.dot(rt(x2r[...]), wbr[...], preferred_element_type=f32)
                 + jnp.dot(rt(x3r[...]), wcr[...], preferred_element_type=f32)
                 + b1r[...])
        u = _elu(jnp.dot(rt(t), w2r[...], preferred_element_type=f32) + b2r[...])
        o_ref[...] = jnp.dot(rt(u), w3r[...], preferred_element_type=f32) + b3r[...]

    return pl.pallas_call(
        body,
        out_shape=jax.ShapeDtypeStruct((B, 1), jnp.float32),
    )(x1, x2, x3, wa, wb, wc, b1, w2, b2, w3, b3)


def _w2perm(w2, m_in, m_out):
    return w2.reshape(128, m_in, m_out).transpose(0, 2, 1).reshape(128, m_out * m_in)


def kernel(x, edge_attr, iso_type_2, iso_type_3, params, edge_index, batch,
           assignment_index_2, edge_index_2, batch_2,
           assignment_index_3, edge_index_3, batch_3):
    p = params
    f32 = jnp.float32

    x_p = _padr(x.astype(f32), NP)
    ea_p = _padr(edge_attr.astype(f32), EP)
    iso2_p = _padr(iso_type_2.astype(f32), NP)
    iso3_p = _padr(iso_type_3.astype(f32), NP)

    src1 = _padi(edge_index[0], EP, 0)
    dst1 = _padi(edge_index[1], EP, N)
    src2 = _padi(edge_index_2[0], EP, 0)
    dst2 = _padi(edge_index_2[1], EP, N)
    src3 = _padi(edge_index_3[0], EP, 0)
    dst3 = _padi(edge_index_3[1], EP, N)
    a2s = _padi(assignment_index_2[0], AP, 0)
    a2d = _padi(assignment_index_2[1], AP, N)
    a3s = _padi(assignment_index_3[0], AP, 0)
    a3d = _padi(assignment_index_3[1], AP, N)
    bt1 = _padi(batch, NP, B).reshape(NP // BN, 1, BN)
    bt2 = _padi(batch_2, NP, B).reshape(NP // BN, 1, BN)
    bt3 = _padi(batch_3, NP, B).reshape(NP // BN, 1, BN)

    def eyep(m):
        return jnp.concatenate([jnp.eye(m, dtype=f32),
                                jnp.zeros((128 - m, m), f32)], axis=0)

    row128 = lambda v: v.reshape(1, -1).astype(f32)

    w2t1 = _w2perm(p['nn1_W2'], 128, 32)
    w2t2 = _w2perm(p['nn2_W2'], 32, 64)
    w2t3 = _w2perm(p['nn3_W2'], 64, 64)
    b2p = lambda b2, mi, mo: b2.reshape(mi, mo).transpose(1, 0).reshape(1, mo * mi)
    b2m1 = b2p(p['nn1_b2'], 128, 32)
    b2m2 = b2p(p['nn2_b2'], 32, 64)
    b2m3 = b2p(p['nn3_b2'], 64, 64)
    sel1 = jnp.kron(jnp.eye(32, dtype=f32), jnp.ones((128, 1), f32))
    sel2 = jnp.kron(jnp.eye(64, dtype=f32), jnp.ones((32, 1), f32))
    sel3 = jnp.kron(jnp.eye(64, dtype=f32), jnp.ones((64, 1), f32))
    tm2 = jnp.tile(jnp.eye(32, dtype=f32), (1, 64))
    tm3 = jnp.tile(jnp.eye(64, dtype=f32), (1, 64))

    padw = lambda w, r: jnp.concatenate(
        [w.astype(f32), jnp.zeros((r - w.shape[0], w.shape[1]), f32)], axis=0)
    w4r, w4n = padw(p['W4r'], 128), padw(p['W4n'], 128)
    w6r, w6n = padw(p['W6r'], 256), padw(p['W6n'], 256)
    w5r, w5n = padw(p['W5r'], 128), padw(p['W5n'], 128)
    w7r, w7n = padw(p['W7r'], 128), padw(p['W7n'], 128)
    root2p, root3p = padw(p['root2'], 128), padw(p['root3'], 128)

    rt = lambda a: a.astype(jnp.bfloat16).astype(f32)
    fc1s = rt(p['fc1W'][:192]) + rt(p['fc1W'][192:])
    fca, fcb, fcc = fc1s[0:64], fc1s[64:128], fc1s[128:192]

    ech = EP // NW // LCH
    ach = AP // NW // LCH
    xs1 = _sc_gather(x_p, src1, ech)
    msg1 = _nnconv_msgs(ea_p, xs1, p['nn1_W1'], row128(p['nn1_b1']), w2t1, b2m1, sel1, None, 128, 32)
    parts1 = _sc_scatter_add(msg1, dst1, ech)
    h1, _ = _node_affine(x_p, parts1, p['root1'], eyep(32), row128(p['bias1']), bt1, False)

    xs2 = _sc_gather(h1, src1, ech)
    msg2 = _nnconv_msgs(ea_p, xs2, p['nn2_W1'], row128(p['nn2_b1']), w2t2, b2m2, sel2, tm2, 32, 64)
    parts2 = _sc_scatter_add(msg2, dst1, ech)
    h2, _ = _node_affine(h1, parts2, root2p, eyep(64), row128(p['bias2']), bt1, False)

    xs3 = _sc_gather(h2, src1, ech)
    msg3 = _nnconv_msgs(ea_p, xs3, p['nn3_W1'], row128(p['nn3_b1']), w2t3, b2m3, sel3, tm3, 64, 64)
    parts3 = _sc_scatter_add(msg3, dst1, ech)
    h3cat, x1 = _node_affine(h2, parts3, root3p, eyep(64), row128(p['bias3']), bt1, True)

    pp2 = _sc_gather_scatter(h3cat, a2s, a2d, ach)
    h2cat = _pool_concat(pp2, iso2_p, 128)
    g4 = _sc_gather_scatter(h2cat, src2, dst2, ech)
    h4, _ = _node_affine(h2cat, g4, w4r, w4n, row128(p['b4']), bt2, False, cast_wn=True)
    g5 = _sc_gather_scatter(h4, src2, dst2, ech)
    h5, x2 = _node_affine(h4, g5, w5r, w5n, row128(p['b5']), bt2, False, cast_wn=True)

    pp3 = _sc_gather_scatter(h3cat, a3s, a3d, ach)
    h3cat2 = _pool_concat(pp3, iso3_p, 256)
    h3a, h3b = h3cat2[:, :128], h3cat2[:, 128:]
    g6a = _sc_gather_scatter(h3a, src3, dst3, ech)
    g6b = _sc_gather_scatter(h3b, src3, dst3, ech)
    h6, _ = _node_affine(h3cat2, g6a, w6r, w6n[:128], row128(p['b6']), bt3,
                         False, parts_b=g6b, wn_b=w6n[128:], cast_wn=True)
    g7 = _sc_gather_scatter(h6, src3, dst3, ech)
    h7, x3 = _node_affine(h6, g7, w7r, w7n, row128(p['b7']), bt3, False)

    o = _mlp(x1, x2, x3, fca, fcb, fcc, row128(p['fc1b']),
             rt(p['fc2W']), row128(p['fc2b']), rt(p['fc3W']), p['fc3b'].reshape(1, 1))
    return o.reshape(-1)


# speedup vs baseline: 2.5536x; 1.0212x over previous
import functools

import jax
import jax.numpy as jnp
from jax import lax
from jax.experimental import pallas as pl
from jax.experimental.pallas import tpu as pltpu
from jax.experimental.pallas import tpu_sc as plsc

N = 10000
NP = 10240
E = 160000
EP = 163840
A = 30000
AP = 32768
B = 64
BN = 1024
EBLK = 256
NW = 32
LCH = 128


def _padr(a, n):
    pad = jnp.zeros((n - a.shape[0],) + a.shape[1:], a.dtype)
    return jnp.concatenate([a, pad], axis=0)


def _padi(v, n, fill):
    pad = jnp.full((n - v.shape[0],), fill, jnp.int32)
    return jnp.concatenate([v.astype(jnp.int32), pad], axis=0)


def _sc_mesh():
    return plsc.VectorSubcoreMesh(core_axis_name="c", subcore_axis_name="s")


def _zero_vmem(rows_v, d):
    zv = jnp.zeros((16,), jnp.float32)

    def zr(r, c):
        for l in range(d // 16):
            rows_v[r, pl.ds(l * 16, 16)] = zv
        return c

    lax.fori_loop(0, LCH, zr, 0)


def _sc_gather(table, idx, nch):
    npn, d = table.shape
    epn = idx.shape[0]

    @functools.partial(
        pl.kernel,
        out_type=jax.ShapeDtypeStruct((epn, d), jnp.float32),
        mesh=_sc_mesh(),
        scratch_types=[
            pltpu.VMEM((nch, LCH), jnp.int32),
            pltpu.VMEM((LCH, d), jnp.float32),
            pltpu.SemaphoreType.DMA,
        ],
    )
    def k(table_hbm, idx_hbm, out_hbm, idx_v, rows_v, sem):
        wid = lax.axis_index("s") * 2 + lax.axis_index("c")
        base = wid * (nch * LCH)
        pltpu.sync_copy(idx_hbm.at[pl.ds(wid * nch, nch)], idx_v)

        def body(j, c):
            off = base + j * LCH
            pltpu.async_copy(table_hbm.at[idx_v.at[j]], rows_v, sem).wait()
            pltpu.sync_copy(rows_v, out_hbm.at[pl.ds(off, LCH)])
            return c

        lax.fori_loop(0, nch, body, 0)

    return k(table, idx.reshape(epn // LCH, LCH))


def _sc_segsum(rows_or_table, src_idx, dst_idx, nch, indirect_src):
    d = rows_or_table.shape[1]
    nst = NP // 16 // LCH

    scratch = [
        pltpu.VMEM((nch, LCH), jnp.int32),
        pltpu.VMEM((LCH, d), jnp.float32),
        pltpu.VMEM_SHARED((NP, d), jnp.float32),
        pltpu.SemaphoreType.DMA,
    ]
    if indirect_src:
        scratch.insert(0, pltpu.VMEM((nch, LCH), jnp.int32))

    @functools.partial(
        pl.kernel,
        out_type=jax.ShapeDtypeStruct((2, NP, d), jnp.float32),
        mesh=_sc_mesh(),
        scratch_types=scratch,
    )
    def k(rows_hbm, src_hbm, dst_hbm, out_hbm, *refs):
        if indirect_src:
            src_v, dst_v, rows_v, agg_sh, sem = refs
        else:
            dst_v, rows_v, agg_sh, sem = refs
            src_v = None
        cid = lax.axis_index("c")
        sid = lax.axis_index("s")
        wid = sid * 2 + cid
        base = wid * (nch * LCH)
        srow = sid * (NP // 16)
        pltpu.sync_copy(dst_hbm.at[pl.ds(wid * nch, nch)], dst_v)
        if indirect_src:
            pltpu.sync_copy(src_hbm.at[pl.ds(wid * nch, nch)], src_v)

        _zero_vmem(rows_v, d)
        for jj in range(nst):
            pltpu.sync_copy(rows_v, agg_sh.at[pl.ds(srow + jj * LCH, LCH)])
        plsc.subcore_barrier()

        def body(j, c):
            off = base + j * LCH
            if indirect_src:
                pltpu.async_copy(rows_hbm.at[src_v.at[j]], rows_v, sem).wait()
            else:
                pltpu.sync_copy(rows_hbm.at[pl.ds(off, LCH)], rows_v)
            pltpu.sync_copy(rows_v, agg_sh.at[dst_v.at[j]], add=True)
            return c

        lax.fori_loop(0, nch, body, 0)
        plsc.subcore_barrier()

        for jj in range(nst):
            r0 = srow + jj * LCH
            pltpu.sync_copy(agg_sh.at[pl.ds(r0, LCH)], rows_v)
            pltpu.sync_copy(rows_v, out_hbm.at[cid, pl.ds(r0, LCH)])

    nblk = src_idx.shape[0] // LCH
    return k(rows_or_table, src_idx.reshape(nblk, LCH), dst_idx.reshape(nblk, LCH))


def _sc_scatter_add(rows, dst_idx, nch):
    return _sc_segsum(rows, dst_idx, dst_idx, nch, indirect_src=False)


def _sc_gather_scatter(table, src_idx, dst_idx, nch):
    return _sc_segsum(table, src_idx, dst_idx, nch, indirect_src=True)


def _elu(v):
    return jnp.where(v > 0, v, jnp.exp(jnp.minimum(v, 0.0)) - 1.0)


def _nnconv_msgs(ea, xs, w1, b1, w2t, b2m, sel, tilemat, m_in, m_out):
    bf = jnp.bfloat16
    f32 = jnp.float32

    def body(ea_ref, xs_ref, w1_ref, b1_ref, w2t_ref, b2p_ref, sel_ref,
             *rest):
        tm_ref, out_ref = (rest if m_in < 128 else (None, rest[0]))
        h = jnp.maximum(
            jnp.dot(ea_ref[...].astype(bf), w1_ref[...].astype(bf),
                    preferred_element_type=f32)
            + b1_ref[...], 0.0)
        xs_m = xs_ref[...] if m_in == 128 else xs_ref[...][:, :m_in]
        wet = jnp.dot(h.astype(bf), w2t_ref[...].astype(bf),
                      preferred_element_type=f32) + b2p_ref[...]
        wet = wet.astype(bf).astype(f32)
        xs_b = xs_m.astype(bf).astype(f32)
        if m_in == 128:
            xst = jnp.tile(xs_b, (1, m_out))
        else:
            xst = jnp.dot(xs_b, tm_ref[...], preferred_element_type=f32)
        r = wet * xst
        msg = jnp.dot(r, sel_ref[...], preferred_element_type=f32)
        if m_out < 128:
            msg = jnp.concatenate(
                [msg, jnp.zeros((EBLK, 128 - m_out), f32)], axis=1)
        out_ref[...] = msg

    in_specs = [
        pl.BlockSpec((EBLK, 7), lambda i: (i, 0)),
        pl.BlockSpec((EBLK, 128), lambda i: (i, 0)),
        pl.BlockSpec((7, 128), lambda i: (0, 0)),
        pl.BlockSpec((1, 128), lambda i: (0, 0)),
        pl.BlockSpec((128, m_out * m_in), lambda i: (0, 0)),
        pl.BlockSpec((1, m_out * m_in), lambda i: (0, 0)),
        pl.BlockSpec((m_out * m_in, m_out), lambda i: (0, 0)),
    ]
    args = [ea, xs, w1, b1, w2t, b2m, sel]
    if m_in < 128:
        in_specs.append(pl.BlockSpec((m_in, m_out * m_in), lambda i: (0, 0)))
        args.append(tilemat)

    return pl.pallas_call(
        body,
        grid=(EP // EBLK,),
        in_specs=in_specs,
        out_specs=pl.BlockSpec((EBLK, 128), lambda i: (i, 0)),
        out_shape=jax.ShapeDtypeStruct((EP, 128), jnp.float32),
    )(*args)


def _node_affine(x, parts, wr, wn, b, batch3, emit_cat, parts_b=None, wn_b=None,
                 cast_wn=False):
    di = x.shape[1]
    dm = parts.shape[2]
    do = wr.shape[1]
    two_parts = parts_b is not None

    def body(*refs):
        if two_parts:
            (x_ref, p_ref, pb_ref, wr_ref, wn_ref, wnb_ref, b_ref, bt_ref,
             h_ref, pool_ref) = refs
        else:
            x_ref, p_ref, wr_ref, wn_ref, b_ref, bt_ref, h_ref, pool_ref = refs
        bf = jnp.bfloat16
        f32 = jnp.float32
        agg = p_ref[0] + p_ref[1]
        if cast_wn:
            aggd = jnp.dot(agg.astype(bf), wn_ref[...].astype(bf),
                           preferred_element_type=f32)
        else:
            aggd = jnp.dot(agg, wn_ref[...], preferred_element_type=f32)
        h = (jnp.dot(x_ref[...].astype(bf), wr_ref[...].astype(bf),
                     preferred_element_type=f32)
             + aggd + b_ref[...])
        if two_parts:
            aggb = pb_ref[0] + pb_ref[1]
            h = h + jnp.dot(aggb.astype(bf), wnb_ref[...].astype(bf),
                            preferred_element_type=f32)
        h = _elu(h)
        if emit_cat:
            pad_col = (lax.broadcasted_iota(jnp.int32, (BN, 128 - do), 1) == 0
                       ).astype(jnp.float32)
        else:
            pad_col = jnp.zeros((BN, 128 - do), jnp.float32)
        h_ref[...] = jnp.concatenate([h, pad_col], axis=1)
        bt = bt_ref[0]
        oh = (lax.broadcasted_iota(jnp.int32, (B, BN), 0) == bt).astype(jnp.float32)
        pool_blk = jnp.dot(oh, h, preferred_element_type=jnp.float32)

        @pl.when(pl.program_id(0) == 0)
        def _():
            pool_ref[...] = jnp.zeros_like(pool_ref)

        pool_ref[...] += pool_blk

    in_specs = [
        pl.BlockSpec((BN, di), lambda i: (i, 0)),
        pl.BlockSpec((2, BN, dm), lambda i: (0, i, 0)),
    ]
    args = [x, parts]
    if two_parts:
        in_specs.append(pl.BlockSpec((2, BN, 128), lambda i: (0, i, 0)))
        args.append(parts_b)
    in_specs.append(pl.BlockSpec((di, do), lambda i: (0, 0)))
    args.append(wr)
    in_specs.append(pl.BlockSpec((dm, do), lambda i: (0, 0)))
    args.append(wn)
    if two_parts:
        in_specs.append(pl.BlockSpec((128, do), lambda i: (0, 0)))
        args.append(wn_b)
    in_specs.append(pl.BlockSpec((1, do), lambda i: (0, 0)))
    args.append(b)
    in_specs.append(pl.BlockSpec((1, 1, BN), lambda i: (i, 0, 0)))
    args.append(batch3)

    return pl.pallas_call(
        body,
        grid=(NP // BN,),
        in_specs=in_specs,
        out_specs=[
            pl.BlockSpec((BN, 128), lambda i: (i, 0)),
            pl.BlockSpec((B, do), lambda i: (0, 0)),
        ],
        out_shape=[
            jax.ShapeDtypeStruct((NP, 128), jnp.float32),
            jax.ShapeDtypeStruct((B, do), jnp.float32),
        ],
    )(*args)


def _pool_concat(parts, iso, outw):
    ni = iso.shape[1]

    def body(p_ref, iso_ref, o_ref):
        s = p_ref[0] + p_ref[1]
        cnt = s[:, 64:65]
        pool = s[:, :64] / jnp.maximum(cnt, 1.0)
        pad = jnp.zeros((BN, outw - 64 - ni), jnp.float32)
        o_ref[...] = jnp.concatenate([pool, iso_ref[...], pad], axis=1)

    return pl.pallas_call(
        body,
        grid=(NP // BN,),
        in_specs=[
            pl.BlockSpec((2, BN, 128), lambda i: (0, i, 0)),
            pl.BlockSpec((BN, ni), lambda i: (i, 0)),
        ],
        out_specs=pl.BlockSpec((BN, outw), lambda i: (i, 0)),
        out_shape=jax.ShapeDtypeStruct((NP, outw), jnp.float32),
    )(parts, iso)


def _mlp(x1, x2, x3, wa, wb, wc, b1, w2, b2, w3, b3):
    def body(x1r, x2r, x3r, war, wbr, wcr, b1r, w2r, b2r, w3r, b3r, o_ref):
        f32 = jnp.float32
        rt = lambda a: a.astype(jnp.bfloat16).astype(f32)
        t = _elu(jnp.dot(rt(x1r[...]), war[...], preferred_element_type=f32)
                 + jnp.dot(rt(x2r[...]), wbr[...], preferred_element_type=f32)
                 + jnp.dot(rt(x3r[...]), wcr[...], preferred_element_type=f32)
                 + b1r[...])
        u = _elu(jnp.dot(rt(t), w2r[...], preferred_element_type=f32) + b2r[...])
        o_ref[...] = jnp.dot(rt(u), w3r[...], preferred_element_type=f32) + b3r[...]

    return pl.pallas_call(
        body,
        out_shape=jax.ShapeDtypeStruct((B, 1), jnp.float32),
    )(x1, x2, x3, wa, wb, wc, b1, w2, b2, w3, b3)


def _w2perm(w2, m_in, m_out):
    return w2.reshape(128, m_in, m_out).transpose(0, 2, 1).reshape(128, m_out * m_in)


def kernel(x, edge_attr, iso_type_2, iso_type_3, params, edge_index, batch,
           assignment_index_2, edge_index_2, batch_2,
           assignment_index_3, edge_index_3, batch_3):
    p = params
    f32 = jnp.float32

    x_p = _padr(x.astype(f32), NP)
    ea_p = _padr(edge_attr.astype(f32), EP)
    iso2_p = _padr(iso_type_2.astype(f32), NP)
    iso3_p = _padr(iso_type_3.astype(f32), NP)

    src1 = _padi(edge_index[0], EP, 0)
    dst1 = _padi(edge_index[1], EP, N)
    src2 = _padi(edge_index_2[0], EP, 0)
    dst2 = _padi(edge_index_2[1], EP, N)
    src3 = _padi(edge_index_3[0], EP, 0)
    dst3 = _padi(edge_index_3[1], EP, N)
    a2s = _padi(assignment_index_2[0], AP, 0)
    a2d = _padi(assignment_index_2[1], AP, N)
    a3s = _padi(assignment_index_3[0], AP, 0)
    a3d = _padi(assignment_index_3[1], AP, N)
    bt1 = _padi(batch, NP, B).reshape(NP // BN, 1, BN)
    bt2 = _padi(batch_2, NP, B).reshape(NP // BN, 1, BN)
    bt3 = _padi(batch_3, NP, B).reshape(NP // BN, 1, BN)

    def eyep(m):
        return jnp.concatenate([jnp.eye(m, dtype=f32),
                                jnp.zeros((128 - m, m), f32)], axis=0)

    row128 = lambda v: v.reshape(1, -1).astype(f32)

    w2t1 = _w2perm(p['nn1_W2'], 128, 32)
    w2t2 = _w2perm(p['nn2_W2'], 32, 64)
    w2t3 = _w2perm(p['nn3_W2'], 64, 64)
    b2p = lambda b2, mi, mo: b2.reshape(mi, mo).transpose(1, 0).reshape(1, mo * mi)
    b2m1 = b2p(p['nn1_b2'], 128, 32)
    b2m2 = b2p(p['nn2_b2'], 32, 64)
    b2m3 = b2p(p['nn3_b2'], 64, 64)
    sel1 = jnp.kron(jnp.eye(32, dtype=f32), jnp.ones((128, 1), f32))
    sel2 = jnp.kron(jnp.eye(64, dtype=f32), jnp.ones((32, 1), f32))
    sel3 = jnp.kron(jnp.eye(64, dtype=f32), jnp.ones((64, 1), f32))
    tm2 = jnp.tile(jnp.eye(32, dtype=f32), (1, 64))
    tm3 = jnp.tile(jnp.eye(64, dtype=f32), (1, 64))

    padw = lambda w, r: jnp.concatenate(
        [w.astype(f32), jnp.zeros((r - w.shape[0], w.shape[1]), f32)], axis=0)
    w4r, w4n = padw(p['W4r'], 128), padw(p['W4n'], 128)
    w6r, w6n = padw(p['W6r'], 256), padw(p['W6n'], 256)
    w5r, w5n = padw(p['W5r'], 128), padw(p['W5n'], 128)
    w7r, w7n = padw(p['W7r'], 128), padw(p['W7n'], 128)
    root2p, root3p = padw(p['root2'], 128), padw(p['root3'], 128)

    rt = lambda a: a.astype(jnp.bfloat16).astype(f32)
    fc1s = rt(p['fc1W'][:192]) + rt(p['fc1W'][192:])
    fca, fcb, fcc = fc1s[0:64], fc1s[64:128], fc1s[128:192]

    ech = EP // NW // LCH
    ach = AP // NW // LCH
    xs1 = _sc_gather(x_p, src1, ech)
    msg1 = _nnconv_msgs(ea_p, xs1, p['nn1_W1'], row128(p['nn1_b1']), w2t1, b2m1, sel1, None, 128, 32)
    parts1 = _sc_scatter_add(msg1, dst1, ech)
    h1, _ = _node_affine(x_p, parts1, p['root1'], eyep(32), row128(p['bias1']), bt1, False)

    xs2 = _sc_gather(h1, src1, ech)
    msg2 = _nnconv_msgs(ea_p, xs2, p['nn2_W1'], row128(p['nn2_b1']), w2t2, b2m2, sel2, tm2, 32, 64)
    parts2 = _sc_scatter_add(msg2, dst1, ech)
    h2, _ = _node_affine(h1, parts2, root2p, eyep(64), row128(p['bias2']), bt1, False)

    xs3 = _sc_gather(h2, src1, ech)
    msg3 = _nnconv_msgs(ea_p, xs3, p['nn3_W1'], row128(p['nn3_b1']), w2t3, b2m3, sel3, tm3, 64, 64)
    parts3 = _sc_scatter_add(msg3, dst1, ech)
    h3cat, x1 = _node_affine(h2, parts3, root3p, eyep(64), row128(p['bias3']), bt1, True)

    pp2 = _sc_gather_scatter(h3cat, a2s, a2d, ach)
    h2cat = _pool_concat(pp2, iso2_p, 128)
    g4 = _sc_gather_scatter(h2cat, src2, dst2, ech)
    h4, _ = _node_affine(h2cat, g4, w4r, w4n, row128(p['b4']), bt2, False, cast_wn=True)
    g5 = _sc_gather_scatter(h4, src2, dst2, ech)
    h5, x2 = _node_affine(h4, g5, w5r, w5n, row128(p['b5']), bt2, False, cast_wn=True)

    pp3 = _sc_gather_scatter(h3cat, a3s, a3d, ach)
    h3cat2 = _pool_concat(pp3, iso3_p, 256)
    h3a, h3b = h3cat2[:, :128], h3cat2[:, 128:]
    g6a = _sc_gather_scatter(h3a, src3, dst3, ech)
    g6b = _sc_gather_scatter(h3b, src3, dst3, ech)
    h6, _ = _node_affine(h3cat2, g6a, w6r, w6n[:128], row128(p['b6']), bt3,
                         False, parts_b=g6b, wn_b=w6n[128:], cast_wn=True)
    g7 = _sc_gather_scatter(h6, src3, dst3, ech)
    h7, x3 = _node_affine(h6, g7, w7r, w7n, row128(p['b7']), bt3, False)

    o = _mlp(x1, x2, x3, fca, fcb, fcc, row128(p['fc1b']),
             rt(p['fc2W']), row128(p['fc2b']), rt(p['fc3W']), p['fc3b'].reshape(1, 1))
    return o.reshape(-1)
